# baseline (speedup 1.0000x reference)
import dataclasses

import jax
import jax.numpy as jnp
from jax import lax
from jax.experimental import pallas as pl
from jax.experimental.pallas import tpu as pltpu
from jax.experimental.pallas import tpu_sc as plsc

NUM_CLASSES = 1000
B0 = 4096
B1 = 20

TC_ROWS = 2048
TC_BLOCK = 128
SC_ROWS = B0 - TC_ROWS

N_WORKERS = 32
ROWS_PER_W = SC_ROWS // N_WORKERS
IDX_PER_W = ROWS_PER_W * B1
CH_D0 = 2
N_BUF = 2
N_CHUNKS = ROWS_PER_W // CH_D0
CH_LOGICAL = CH_D0 * B1
N_GROUPS = (CH_LOGICAL + 15) // 16


def _tc_kernel(idx_ref, out_ref):
    idx = idx_ref[...]
    classes = jax.lax.broadcasted_iota(
        jnp.int32, (TC_BLOCK, B1, NUM_CLASSES), 2)
    out_ref[...] = (idx[:, :, None] == classes).astype(jnp.float32)


def _tc_half(idx):
    return pl.pallas_call(
        _tc_kernel,
        grid=(TC_ROWS // TC_BLOCK,),
        in_specs=[pl.BlockSpec((TC_BLOCK, B1), lambda i: (i, 0))],
        out_specs=pl.BlockSpec(
            (TC_BLOCK, B1, NUM_CLASSES), lambda i: (i, 0, 0)),
        out_shape=jax.ShapeDtypeStruct((TC_ROWS, B1, NUM_CLASSES),
                                       jnp.float32),
    )(idx[:TC_ROWS])


def _sc_kernel(idx_hbm, out_hbm, buf, idx_v, sems):
    cid = lax.axis_index("c")
    sid = lax.axis_index("s")
    wid = sid * 2 + cid

    iota = lax.iota(jnp.int32, 16)
    zeros16 = jnp.zeros((16,), jnp.float32)
    ones16 = jnp.ones((16,), jnp.float32)

    pltpu.sync_copy(
        idx_hbm.at[pl.ds(TC_ROWS * B1 + wid * IDX_PER_W, IDX_PER_W)], idx_v)

    def zero_row(r, carry):
        b = r // (CH_D0 * B1)
        a0 = lax.rem(r, CH_D0 * B1) // B1
        a1 = lax.rem(r, B1)
        for k in range(62):
            buf[b, a0, a1, pl.ds(k * 16, 16)] = zeros16
        buf[b, a0, a1, pl.ds(NUM_CLASSES - 16, 16)] = zeros16
        return carry

    lax.fori_loop(0, N_BUF * CH_LOGICAL, zero_row, 0)

    def scatter_chunk(b, c, value16):
        for k in range(N_GROUPS):
            g = iota + k * 16
            mask = g < CH_LOGICAL
            i0 = g // B1
            i1 = lax.rem(g, B1)
            i2 = idx_v[pl.ds(c * CH_LOGICAL + k * 16, 16)]
            plsc.store_scatter(buf.at[b], [i0, i1, i2], value16, mask=mask)

    def chunk_dma(b, c):
        d0 = wid * ROWS_PER_W + c * CH_D0
        return pltpu.make_async_copy(
            buf.at[b], out_hbm.at[pl.ds(d0, CH_D0)], sems.at[b])

    def chunk_body(c, carry):
        b = lax.rem(c, N_BUF)

        @pl.when(c >= N_BUF)
        def _reclaim():
            chunk_dma(b, c - N_BUF).wait()
            scatter_chunk(b, c - N_BUF, zeros16)

        scatter_chunk(b, c, ones16)
        chunk_dma(b, c).start()
        return carry

    lax.fori_loop(0, N_CHUNKS, chunk_body, 0)
    for j in range(N_BUF):
        c = N_CHUNKS - N_BUF + j
        chunk_dma(c % N_BUF, c).wait()


def _sc_half(idx_flat):
    mesh = plsc.VectorSubcoreMesh(core_axis_name="c", subcore_axis_name="s")
    cp = pltpu.CompilerParams()
    if "needs_layout_passes" in pltpu.CompilerParams.__dataclass_fields__:
        cp = dataclasses.replace(cp, needs_layout_passes=False)
    run = pl.kernel(
        _sc_kernel,
        mesh=mesh,
        compiler_params=cp,
        out_type=jax.ShapeDtypeStruct((SC_ROWS, B1, NUM_CLASSES),
                                      jnp.float32),
        scratch_types=[
            pltpu.VMEM((N_BUF, CH_D0, B1, NUM_CLASSES), jnp.float32),
            pltpu.VMEM((IDX_PER_W,), jnp.int32),
            pltpu.SemaphoreType.DMA((N_BUF,)),
        ],
    )
    return run(idx_flat)


def kernel(inputs):
    idx = inputs.astype(jnp.int32)
    top = _tc_half(idx)
    bottom = _sc_half(idx.reshape(-1))
    return jnp.concatenate([top, bottom], axis=0)


# speedup vs baseline: 1.3234x; 1.3234x over previous
import dataclasses

import jax
import jax.numpy as jnp
from jax import lax
from jax.experimental import pallas as pl
from jax.experimental.pallas import tpu as pltpu
from jax.experimental.pallas import tpu_sc as plsc

NUM_CLASSES = 1000
B0 = 4096
B1 = 20
N_WORKERS = 32
ROWS_PER_W = B0 // N_WORKERS
IDX_PER_W = ROWS_PER_W * B1
CH_D0 = 2
N_BUF = 2
N_CHUNKS = ROWS_PER_W // CH_D0
CH_LOGICAL = CH_D0 * B1
N_GROUPS = (CH_LOGICAL + 15) // 16


def _sc_kernel(idx_hbm, out_hbm, buf, idx_v, sems):
    cid = lax.axis_index("c")
    sid = lax.axis_index("s")
    wid = sid * 2 + cid

    iota = lax.iota(jnp.int32, 16)
    zeros16 = jnp.zeros((16,), jnp.float32)
    ones16 = jnp.ones((16,), jnp.float32)

    pltpu.sync_copy(idx_hbm.at[pl.ds(wid * IDX_PER_W, IDX_PER_W)], idx_v)

    def zero_row(r, carry):
        b = r // (CH_D0 * B1)
        a0 = lax.rem(r, CH_D0 * B1) // B1
        a1 = lax.rem(r, B1)
        for k in range(62):
            buf[b, a0, a1, pl.ds(k * 16, 16)] = zeros16
        buf[b, a0, a1, pl.ds(NUM_CLASSES - 16, 16)] = zeros16
        return carry

    lax.fori_loop(0, N_BUF * CH_LOGICAL, zero_row, 0)

    def scatter_chunk(b, c, value16):
        for k in range(N_GROUPS):
            g = iota + k * 16
            mask = g < CH_LOGICAL
            i0 = g // B1
            i1 = lax.rem(g, B1)
            i2 = idx_v[pl.ds(c * CH_LOGICAL + k * 16, 16)]
            plsc.store_scatter(buf.at[b], [i0, i1, i2], value16, mask=mask)

    def chunk_dma(b, c):
        d0 = wid * ROWS_PER_W + c * CH_D0
        return pltpu.make_async_copy(
            buf.at[b], out_hbm.at[pl.ds(d0, CH_D0)], sems.at[b])

    def chunk_body(c, carry):
        b = lax.rem(c, N_BUF)

        @pl.when(c >= N_BUF)
        def _reclaim():
            chunk_dma(b, c - N_BUF).wait()
            scatter_chunk(b, c - N_BUF, zeros16)

        scatter_chunk(b, c, ones16)
        chunk_dma(b, c).start()
        return carry

    lax.fori_loop(0, N_CHUNKS, chunk_body, 0)
    for j in range(N_BUF):
        c = N_CHUNKS - N_BUF + j
        chunk_dma(c % N_BUF, c).wait()


def kernel(inputs):
    idx = inputs.reshape(-1).astype(jnp.int32)
    mesh = plsc.VectorSubcoreMesh(core_axis_name="c", subcore_axis_name="s")
    cp = pltpu.CompilerParams()
    if "needs_layout_passes" in pltpu.CompilerParams.__dataclass_fields__:
        cp = dataclasses.replace(cp, needs_layout_passes=False)
    run = pl.kernel(
        _sc_kernel,
        mesh=mesh,
        compiler_params=cp,
        out_type=jax.ShapeDtypeStruct((B0, B1, NUM_CLASSES), jnp.float32),
        scratch_types=[
            pltpu.VMEM((N_BUF, CH_D0, B1, NUM_CLASSES), jnp.float32),
            pltpu.VMEM((IDX_PER_W,), jnp.int32),
            pltpu.SemaphoreType.DMA((N_BUF,)),
        ],
    )
    return run(idx)
